# baseline (speedup 1.0000x reference)
import jax
import jax.numpy as jnp
from jax import lax
from jax.experimental import pallas as pl
from jax.experimental.pallas import tpu as pltpu
from jax.experimental.pallas import tpu_sc as plsc

VOCAB = 1000000
EMBED = 64
HID = 64
B = 4096
L = 200

NUM_CORES = 2
NUM_SUBCORES = 16
NUM_WORKERS = NUM_CORES * NUM_SUBCORES
BPW = B // NUM_WORKERS

CHUNKS = ((0, 128), (128, 72))


TPAD = 128


def _pool_body(x_hbm, table_hbm, out_hbm, idx_v, buf0, buf1, acc_v, sem0, sem1):
    cid = lax.axis_index("c")
    sid = lax.axis_index("s")
    wid = sid * NUM_CORES + cid
    base = wid * BPW

    pltpu.sync_copy(x_hbm.at[pl.ds(base, BPW)], idx_v)

    bufs = (buf0, buf1)
    sems = (sem0, sem1)

    def issue(i, b):
        for off, n in CHUNKS:
            pltpu.async_copy(
                table_hbm.at[idx_v.at[i, pl.ds(off, n)]],
                bufs[b].at[pl.ds(off, n)],
                sems[b],
            )

    def drain(b):
        pltpu.make_async_copy(table_hbm.at[pl.ds(0, L)], bufs[b], sems[b]).wait()

    issue(0, 0)
    issue(1, 1)

    def outer(g, carry):
        for b in range(2):
            i = g * 2 + b
            drain(b)

            @pl.when(i + 2 < BPW)
            def _():
                issue(i + 2, b)

            buf = bufs[b]

            def red(j, accs):
                a = list(accs)
                for k in range(8):
                    r = j * 8 + k
                    for c in range(4):
                        a[c] = a[c] + buf[r, pl.ds(c * 16, 16)]
                return tuple(a)

            accs = lax.fori_loop(
                0, L // 8, red, tuple(jnp.zeros((16,), jnp.float32) for _ in range(4))
            )
            for c in range(4):
                acc_v[i, pl.ds(c * 16, 16)] = accs[c]
        return carry

    lax.fori_loop(0, BPW // 2, outer, 0)

    pltpu.sync_copy(acc_v, out_hbm.at[pl.ds(base, BPW)])


def _pooled_sums(x, table):
    mesh = plsc.VectorSubcoreMesh(core_axis_name="c", subcore_axis_name="s")
    return pl.kernel(
        _pool_body,
        out_type=jax.ShapeDtypeStruct((B, EMBED), jnp.float32),
        mesh=mesh,
        compiler_params=pltpu.CompilerParams(use_tc_tiling_on_sc=False),
        scratch_types=[
            pltpu.VMEM((BPW, L), jnp.int32),
            pltpu.VMEM((L, TPAD), jnp.float32),
            pltpu.VMEM((L, TPAD), jnp.float32),
            pltpu.VMEM((BPW, EMBED), jnp.float32),
            pltpu.SemaphoreType.DMA,
            pltpu.SemaphoreType.DMA,
        ],
    )(x, table)


TCOLS = 2048


def _detile_body(tt_ref, out_ref):
    eye = jnp.eye(EMBED, dtype=jnp.float32)
    t = jax.lax.dot_general(
        tt_ref[...], eye, (((0,), (0,)), ((), ())),
        preferred_element_type=jnp.float32,
        precision=jax.lax.Precision.HIGHEST,
    )
    out_ref[...] = jnp.concatenate([t, jnp.zeros_like(t)], axis=1)


def _detile(table_t):
    return pl.pallas_call(
        _detile_body,
        grid=((VOCAB + TCOLS - 1) // TCOLS,),
        in_specs=[pl.BlockSpec((EMBED, TCOLS), lambda i: (0, i))],
        out_specs=pl.BlockSpec((TCOLS, TPAD), lambda i: (i, 0)),
        out_shape=jax.ShapeDtypeStruct((VOCAB, TPAD), jnp.float32),
    )(table_t)


def _mlp_body(sum_ref, wc_ref, bc_ref, wh_ref, bh_ref, out_ref):
    pooled = sum_ref[...] * (1.0 / L)
    h = jnp.dot(pooled, wc_ref[...], preferred_element_type=jnp.float32)
    h = jnp.maximum(h + bc_ref[...], 0.0)
    out_ref[...] = (
        jnp.dot(h, wh_ref[...], preferred_element_type=jnp.float32) + bh_ref[...]
    )


def _mlp(sums, wc_t, bc, wh_t, bh):
    return pl.pallas_call(
        _mlp_body,
        out_shape=jax.ShapeDtypeStruct((B, 16), jnp.float32),
    )(sums, wc_t, bc, wh_t, bh)


def kernel(x, table, W_common, b_common, W_stars, b_stars, W_useful, b_useful,
           W_funny, b_funny, W_cool, b_cool):
    table_padded = _detile(table.T)
    sums = _pooled_sums(x, table_padded)
    wh = jnp.concatenate([W_stars, W_useful, W_funny, W_cool], axis=0)
    bh = jnp.concatenate([b_stars, b_useful, b_funny, b_cool], axis=0)
    logits = _mlp(
        sums,
        W_common.T,
        b_common.reshape(1, HID),
        wh.T,
        bh.reshape(1, 16),
    )
    return (logits[:, 0:4], logits[:, 4:8], logits[:, 8:12], logits[:, 12:16])


# speedup vs baseline: 1.2228x; 1.2228x over previous
import jax
import jax.numpy as jnp
from jax import lax
from jax.experimental import pallas as pl
from jax.experimental.pallas import tpu as pltpu
from jax.experimental.pallas import tpu_sc as plsc

VOCAB = 1000000
EMBED = 64
HID = 64
B = 4096
L = 200

NUM_CORES = 2
NUM_SUBCORES = 16
NUM_WORKERS = NUM_CORES * NUM_SUBCORES
BPW = B // NUM_WORKERS

CHUNKS = ((0, 128), (128, 72))


TPAD = 128


def _pool_body(x_hbm, table_hbm, out_hbm, idx_v, buf0, buf1, acc_v, sem0, sem1):
    cid = lax.axis_index("c")
    sid = lax.axis_index("s")
    wid = sid * NUM_CORES + cid
    base = wid * BPW

    pltpu.sync_copy(x_hbm.at[pl.ds(base, BPW)], idx_v)

    bufs = (buf0, buf1)
    sems = (sem0, sem1)

    def issue(i, b):
        for off, n in CHUNKS:
            pltpu.async_copy(
                table_hbm.at[idx_v.at[i, pl.ds(off, n)]],
                bufs[b].at[pl.ds(off, n)],
                sems[b],
            )

    def drain(b):
        pltpu.make_async_copy(table_hbm.at[pl.ds(0, L)], bufs[b], sems[b]).wait()

    issue(0, 0)
    issue(1, 1)

    def outer(g, carry):
        for b in range(2):
            i = g * 2 + b
            drain(b)

            @pl.when(i + 2 < BPW)
            def _():
                issue(i + 2, b)

            buf = bufs[b]

            def red(j, accs):
                a = list(accs)
                for k in range(8):
                    r = j * 8 + k
                    for c in range(4):
                        a[c] = a[c] + buf[r, pl.ds(c * 16, 16)]
                return tuple(a)

            accs = lax.fori_loop(
                0, L // 8, red, tuple(jnp.zeros((16,), jnp.float32) for _ in range(4))
            )
            for c in range(4):
                acc_v[i, pl.ds(c * 16, 16)] = accs[c]
        return carry

    lax.fori_loop(0, BPW // 2, outer, 0)

    pltpu.sync_copy(acc_v, out_hbm.at[pl.ds(base, BPW)])


def _pooled_sums(x, table):
    mesh = plsc.VectorSubcoreMesh(core_axis_name="c", subcore_axis_name="s")
    return pl.kernel(
        _pool_body,
        out_type=jax.ShapeDtypeStruct((B, EMBED), jnp.float32),
        mesh=mesh,
        compiler_params=pltpu.CompilerParams(use_tc_tiling_on_sc=False),
        scratch_types=[
            pltpu.VMEM((BPW, L), jnp.int32),
            pltpu.VMEM((L, TPAD), jnp.float32),
            pltpu.VMEM((L, TPAD), jnp.float32),
            pltpu.VMEM((BPW, EMBED), jnp.float32),
            pltpu.SemaphoreType.DMA,
            pltpu.SemaphoreType.DMA,
        ],
    )(x, table)


TCOLS = 2048


def _detile_body(tt_ref, out_ref):
    t = tt_ref[...].T
    out_ref[...] = jnp.concatenate([t, jnp.zeros_like(t)], axis=1)


def _detile(table_t):
    return pl.pallas_call(
        _detile_body,
        grid=((VOCAB + TCOLS - 1) // TCOLS,),
        in_specs=[pl.BlockSpec((EMBED, TCOLS), lambda i: (0, i))],
        out_specs=pl.BlockSpec((TCOLS, TPAD), lambda i: (i, 0)),
        out_shape=jax.ShapeDtypeStruct((VOCAB, TPAD), jnp.float32),
    )(table_t)


def _mlp_body(sum_ref, wc_ref, bc_ref, wh_ref, bh_ref, out_ref):
    pooled = sum_ref[...] * (1.0 / L)
    h = jnp.dot(pooled, wc_ref[...], preferred_element_type=jnp.float32)
    h = jnp.maximum(h + bc_ref[...], 0.0)
    out_ref[...] = (
        jnp.dot(h, wh_ref[...], preferred_element_type=jnp.float32) + bh_ref[...]
    )


def _mlp(sums, wc_t, bc, wh_t, bh):
    return pl.pallas_call(
        _mlp_body,
        out_shape=jax.ShapeDtypeStruct((B, 16), jnp.float32),
    )(sums, wc_t, bc, wh_t, bh)


def kernel(x, table, W_common, b_common, W_stars, b_stars, W_useful, b_useful,
           W_funny, b_funny, W_cool, b_cool):
    table_padded = _detile(table.T)
    sums = _pooled_sums(x, table_padded)
    wh = jnp.concatenate([W_stars, W_useful, W_funny, W_cool], axis=0)
    bh = jnp.concatenate([b_stars, b_useful, b_funny, b_cool], axis=0)
    logits = _mlp(
        sums,
        W_common.T,
        b_common.reshape(1, HID),
        wh.T,
        bh.reshape(1, 16),
    )
    return (logits[:, 0:4], logits[:, 4:8], logits[:, 8:12], logits[:, 12:16])


# speedup vs baseline: 1.7273x; 1.4126x over previous
import jax
import jax.numpy as jnp
from jax import lax
from jax.experimental import pallas as pl
from jax.experimental.pallas import tpu as pltpu
from jax.experimental.pallas import tpu_sc as plsc

VOCAB = 1000000
EMBED = 64
HID = 64
B = 4096
L = 200

NUM_CORES = 2
NUM_SUBCORES = 16
NUM_WORKERS = NUM_CORES * NUM_SUBCORES
BPW = B // NUM_WORKERS

CHUNKS = ((0, 128), (128, 72))


TPAD = 128


def _pool_body(x_hbm, table_hbm, out_hbm, idx_v, buf0, buf1, acc_v, sem0, sem1):
    cid = lax.axis_index("c")
    sid = lax.axis_index("s")
    wid = sid * NUM_CORES + cid
    base = wid * BPW

    pltpu.sync_copy(x_hbm.at[pl.ds(base, BPW)], idx_v)

    bufs = (buf0, buf1)
    sems = (sem0, sem1)

    def issue(i, b):
        for off, n in CHUNKS:
            pltpu.async_copy(
                table_hbm.at[idx_v.at[i, pl.ds(off, n)]],
                bufs[b].at[pl.ds(off, n)],
                sems[b],
            )

    def drain(b):
        pltpu.make_async_copy(table_hbm.at[pl.ds(0, L)], bufs[b], sems[b]).wait()

    issue(0, 0)
    issue(1, 1)

    def outer(g, carry):
        for b in range(2):
            i = g * 2 + b
            drain(b)

            @pl.when(i + 2 < BPW)
            def _():
                issue(i + 2, b)

            buf = bufs[b]

            def red(j, accs):
                a = list(accs)
                for k in range(8):
                    r = j * 8 + k
                    for c in range(4):
                        a[c] = a[c] + buf[r, pl.ds(c * 16, 16)]
                return tuple(a)

            accs = lax.fori_loop(
                0, L // 8, red, tuple(jnp.zeros((16,), jnp.float32) for _ in range(4))
            )
            for c in range(4):
                acc_v[i, pl.ds(c * 16, 16)] = accs[c]
        return carry

    lax.fori_loop(0, BPW // 2, outer, 0)

    pltpu.sync_copy(acc_v, out_hbm.at[pl.ds(base, BPW)])


def _pooled_sums(x, table):
    mesh = plsc.VectorSubcoreMesh(core_axis_name="c", subcore_axis_name="s")
    return pl.kernel(
        _pool_body,
        out_type=jax.ShapeDtypeStruct((B, EMBED), jnp.float32),
        mesh=mesh,
        compiler_params=pltpu.CompilerParams(use_tc_tiling_on_sc=False),
        scratch_types=[
            pltpu.VMEM((BPW, L), jnp.int32),
            pltpu.VMEM((L, TPAD), jnp.float32),
            pltpu.VMEM((L, TPAD), jnp.float32),
            pltpu.VMEM((BPW, EMBED), jnp.float32),
            pltpu.SemaphoreType.DMA,
            pltpu.SemaphoreType.DMA,
        ],
    )(x, table)


TCOLS = 8192


def _detile_body(tt_ref, out_ref):
    t = tt_ref[...].T
    out_ref[...] = jnp.concatenate([t, jnp.zeros_like(t)], axis=1)


def _detile(table_t):
    return pl.pallas_call(
        _detile_body,
        grid=((VOCAB + TCOLS - 1) // TCOLS,),
        in_specs=[pl.BlockSpec((EMBED, TCOLS), lambda i: (0, i))],
        out_specs=pl.BlockSpec((TCOLS, TPAD), lambda i: (i, 0)),
        out_shape=jax.ShapeDtypeStruct((VOCAB, TPAD), jnp.float32),
    )(table_t)


def _mlp_body(sum_ref, wc_ref, bc_ref, wh_ref, bh_ref, out_ref):
    pooled = sum_ref[...] * (1.0 / L)
    h = jnp.dot(pooled, wc_ref[...], preferred_element_type=jnp.float32)
    h = jnp.maximum(h + bc_ref[...], 0.0)
    out_ref[...] = (
        jnp.dot(h, wh_ref[...], preferred_element_type=jnp.float32) + bh_ref[...]
    )


def _mlp(sums, wc_t, bc, wh_t, bh):
    return pl.pallas_call(
        _mlp_body,
        out_shape=jax.ShapeDtypeStruct((B, 16), jnp.float32),
    )(sums, wc_t, bc, wh_t, bh)


def kernel(x, table, W_common, b_common, W_stars, b_stars, W_useful, b_useful,
           W_funny, b_funny, W_cool, b_cool):
    table_padded = _detile(table.T)
    sums = _pooled_sums(x, table_padded)
    wh = jnp.concatenate([W_stars, W_useful, W_funny, W_cool], axis=0)
    bh = jnp.concatenate([b_stars, b_useful, b_funny, b_cool], axis=0)
    logits = _mlp(
        sums,
        W_common.T,
        b_common.reshape(1, HID),
        wh.T,
        bh.reshape(1, 16),
    )
    return (logits[:, 0:4], logits[:, 4:8], logits[:, 8:12], logits[:, 12:16])


# speedup vs baseline: 1.8033x; 1.0440x over previous
import jax
import jax.numpy as jnp
from jax import lax
from jax.experimental import pallas as pl
from jax.experimental.pallas import tpu as pltpu
from jax.experimental.pallas import tpu_sc as plsc

VOCAB = 1000000
EMBED = 64
HID = 64
B = 4096
L = 200

NUM_CORES = 2
NUM_SUBCORES = 16
NUM_WORKERS = NUM_CORES * NUM_SUBCORES
BPW = B // NUM_WORKERS

CHUNKS = ((0, 128), (128, 72))


TPAD = 128


def _pool_body(x_hbm, table_hbm, out_hbm, idx_v, buf0, buf1, acc_v, sem0, sem1):
    cid = lax.axis_index("c")
    sid = lax.axis_index("s")
    wid = sid * NUM_CORES + cid
    base = wid * BPW

    pltpu.sync_copy(x_hbm.at[pl.ds(base, BPW)], idx_v)

    bufs = (buf0, buf1)
    sems = (sem0, sem1)

    def issue(i, b):
        for off, n in CHUNKS:
            pltpu.async_copy(
                table_hbm.at[idx_v.at[i, pl.ds(off, n)]],
                bufs[b].at[pl.ds(off, n)],
                sems[b],
            )

    def drain(b):
        pltpu.make_async_copy(table_hbm.at[pl.ds(0, L)], bufs[b], sems[b]).wait()

    issue(0, 0)
    issue(1, 1)

    def outer(g, carry):
        for b in range(2):
            i = g * 2 + b
            drain(b)

            @pl.when(i + 2 < BPW)
            def _():
                issue(i + 2, b)

            buf = bufs[b]

            def red(j, accs):
                a = list(accs)
                for k in range(8):
                    r = j * 8 + k
                    for c in range(4):
                        a[c] = a[c] + buf[r, pl.ds(c * 16, 16)]
                return tuple(a)

            accs = lax.fori_loop(
                0, L // 8, red, tuple(jnp.zeros((16,), jnp.float32) for _ in range(4))
            )
            for c in range(4):
                acc_v[i, pl.ds(c * 16, 16)] = accs[c]
        return carry

    lax.fori_loop(0, BPW // 2, outer, 0)

    pltpu.sync_copy(acc_v, out_hbm.at[pl.ds(base, BPW)])


def _pooled_sums(x, table):
    mesh = plsc.VectorSubcoreMesh(core_axis_name="c", subcore_axis_name="s")
    return pl.kernel(
        _pool_body,
        out_type=jax.ShapeDtypeStruct((B, EMBED), jnp.float32),
        mesh=mesh,
        compiler_params=pltpu.CompilerParams(use_tc_tiling_on_sc=False),
        scratch_types=[
            pltpu.VMEM((BPW, L), jnp.int32),
            pltpu.VMEM((L, TPAD), jnp.float32),
            pltpu.VMEM((L, TPAD), jnp.float32),
            pltpu.VMEM((BPW, EMBED), jnp.float32),
            pltpu.SemaphoreType.DMA,
            pltpu.SemaphoreType.DMA,
        ],
    )(x, table)


TCOLS = 16384


def _detile_body(tt_ref, out_ref):
    t = tt_ref[...].T
    out_ref[...] = jnp.concatenate([t, jnp.zeros_like(t)], axis=1)


def _detile(table_t):
    return pl.pallas_call(
        _detile_body,
        grid=((VOCAB + TCOLS - 1) // TCOLS,),
        in_specs=[pl.BlockSpec((EMBED, TCOLS), lambda i: (0, i))],
        out_specs=pl.BlockSpec((TCOLS, TPAD), lambda i: (i, 0)),
        out_shape=jax.ShapeDtypeStruct((VOCAB, TPAD), jnp.float32),
    )(table_t)


def _mlp_body(sum_ref, wc_ref, bc_ref, wh_ref, bh_ref, out_ref):
    pooled = sum_ref[...] * (1.0 / L)
    h = jnp.dot(pooled, wc_ref[...], preferred_element_type=jnp.float32)
    h = jnp.maximum(h + bc_ref[...], 0.0)
    out_ref[...] = (
        jnp.dot(h, wh_ref[...], preferred_element_type=jnp.float32) + bh_ref[...]
    )


def _mlp(sums, wc_t, bc, wh_t, bh):
    return pl.pallas_call(
        _mlp_body,
        out_shape=jax.ShapeDtypeStruct((B, 16), jnp.float32),
    )(sums, wc_t, bc, wh_t, bh)


def kernel(x, table, W_common, b_common, W_stars, b_stars, W_useful, b_useful,
           W_funny, b_funny, W_cool, b_cool):
    table_padded = _detile(table.T)
    sums = _pooled_sums(x, table_padded)
    wh = jnp.concatenate([W_stars, W_useful, W_funny, W_cool], axis=0)
    bh = jnp.concatenate([b_stars, b_useful, b_funny, b_cool], axis=0)
    logits = _mlp(
        sums,
        W_common.T,
        b_common.reshape(1, HID),
        wh.T,
        bh.reshape(1, 16),
    )
    return (logits[:, 0:4], logits[:, 4:8], logits[:, 8:12], logits[:, 12:16])


# speedup vs baseline: 1.8511x; 1.0265x over previous
import jax
import jax.numpy as jnp
from jax import lax
from jax.experimental import pallas as pl
from jax.experimental.pallas import tpu as pltpu
from jax.experimental.pallas import tpu_sc as plsc

VOCAB = 1000000
EMBED = 64
HID = 64
B = 4096
L = 200

NUM_CORES = 2
NUM_SUBCORES = 16
NUM_WORKERS = NUM_CORES * NUM_SUBCORES
BPW = B // NUM_WORKERS

CHUNKS = ((0, 128), (128, 72))


TPAD = 128


NBUF = 3


def _pool_body(x_hbm, table_hbm, out_hbm, idx_v, buf0, buf1, buf2, acc_v,
               sem0, sem1, sem2):
    cid = lax.axis_index("c")
    sid = lax.axis_index("s")
    wid = sid * NUM_CORES + cid
    base = wid * BPW

    pltpu.sync_copy(x_hbm.at[pl.ds(base, BPW)], idx_v)

    bufs = (buf0, buf1, buf2)
    sems = (sem0, sem1, sem2)

    def issue(i, b):
        for off, n in CHUNKS:
            pltpu.async_copy(
                table_hbm.at[idx_v.at[i, pl.ds(off, n)]],
                bufs[b].at[pl.ds(off, n)],
                sems[b],
            )

    def drain(b):
        pltpu.make_async_copy(table_hbm.at[pl.ds(0, L)], bufs[b], sems[b]).wait()

    for b in range(NBUF):
        issue(b, b)

    def outer(g, carry):
        for b in range(NBUF):
            i = g * NBUF + b

            @pl.when(i < BPW)
            def _():
                drain(b)

                @pl.when(i + NBUF < BPW)
                def _():
                    issue(i + NBUF, b)

                buf = bufs[b]

                def red(j, accs):
                    a = list(accs)
                    for k in range(8):
                        r = j * 8 + k
                        for c in range(4):
                            a[c] = a[c] + buf[r, pl.ds(c * 16, 16)]
                    return tuple(a)

                accs = lax.fori_loop(
                    0, L // 8, red,
                    tuple(jnp.zeros((16,), jnp.float32) for _ in range(4)),
                )
                for c in range(4):
                    acc_v[i, pl.ds(c * 16, 16)] = accs[c]
        return carry

    lax.fori_loop(0, (BPW + NBUF - 1) // NBUF, outer, 0)

    pltpu.sync_copy(acc_v, out_hbm.at[pl.ds(base, BPW)])


def _pooled_sums(x, table):
    mesh = plsc.VectorSubcoreMesh(core_axis_name="c", subcore_axis_name="s")
    return pl.kernel(
        _pool_body,
        out_type=jax.ShapeDtypeStruct((B, EMBED), jnp.float32),
        mesh=mesh,
        compiler_params=pltpu.CompilerParams(use_tc_tiling_on_sc=False),
        scratch_types=[
            pltpu.VMEM((BPW, L), jnp.int32),
            pltpu.VMEM((L, TPAD), jnp.float32),
            pltpu.VMEM((L, TPAD), jnp.float32),
            pltpu.VMEM((L, TPAD), jnp.float32),
            pltpu.VMEM((BPW, EMBED), jnp.float32),
            pltpu.SemaphoreType.DMA,
            pltpu.SemaphoreType.DMA,
            pltpu.SemaphoreType.DMA,
        ],
    )(x, table)


TCOLS = 16384


def _detile_body(tt_ref, out_ref):
    t = tt_ref[...].T
    out_ref[...] = jnp.concatenate([t, jnp.zeros_like(t)], axis=1)


def _detile(table_t):
    return pl.pallas_call(
        _detile_body,
        grid=((VOCAB + TCOLS - 1) // TCOLS,),
        in_specs=[pl.BlockSpec((EMBED, TCOLS), lambda i: (0, i))],
        out_specs=pl.BlockSpec((TCOLS, TPAD), lambda i: (i, 0)),
        out_shape=jax.ShapeDtypeStruct((VOCAB, TPAD), jnp.float32),
    )(table_t)


def _mlp_body(sum_ref, wc_ref, bc_ref, wh_ref, bh_ref, out_ref):
    pooled = sum_ref[...] * (1.0 / L)
    h = jnp.dot(pooled, wc_ref[...], preferred_element_type=jnp.float32)
    h = jnp.maximum(h + bc_ref[...], 0.0)
    out_ref[...] = (
        jnp.dot(h, wh_ref[...], preferred_element_type=jnp.float32) + bh_ref[...]
    )


def _mlp(sums, wc_t, bc, wh_t, bh):
    return pl.pallas_call(
        _mlp_body,
        out_shape=jax.ShapeDtypeStruct((B, 16), jnp.float32),
    )(sums, wc_t, bc, wh_t, bh)


def kernel(x, table, W_common, b_common, W_stars, b_stars, W_useful, b_useful,
           W_funny, b_funny, W_cool, b_cool):
    table_padded = _detile(table.T)
    sums = _pooled_sums(x, table_padded)
    wh = jnp.concatenate([W_stars, W_useful, W_funny, W_cool], axis=0)
    bh = jnp.concatenate([b_stars, b_useful, b_funny, b_cool], axis=0)
    logits = _mlp(
        sums,
        W_common.T,
        b_common.reshape(1, HID),
        wh.T,
        bh.reshape(1, 16),
    )
    return (logits[:, 0:4], logits[:, 4:8], logits[:, 8:12], logits[:, 12:16])
